# baseline (speedup 1.0000x reference)
import jax
import jax.numpy as jnp
from jax import lax
from jax.experimental import pallas as pl
from jax.experimental.pallas import tpu as pltpu
from jax.experimental.pallas import tpu_sc as plsc

B = 512
N = 45000
E = 100000
NC = 2
NS = 16
NW = NC * NS
E_PER_W = 3232
EP = E_PER_W * NW
C = 32
N_CH = E_PER_W // C
VPR = B // 16


def _sc_kernel(probs_t_hbm, child_hbm, parent_hbm, out_hbm,
               ci_v, pi_v, cr0_v, pr0_v, cr1_v, pr1_v, out_v, sem0, sem1):
    wid = lax.axis_index("s") * NC + lax.axis_index("c")
    ebase = wid * E_PER_W

    pltpu.sync_copy(child_hbm.at[pl.ds(ebase, E_PER_W)], ci_v)
    pltpu.sync_copy(parent_hbm.at[pl.ds(ebase, E_PER_W)], pi_v)

    def start_gather(ch, crv, prv, sem):
        pltpu.make_async_copy(
            probs_t_hbm.at[ci_v.at[pl.ds(ch * C, C)]], crv, sem).start()
        pltpu.make_async_copy(
            probs_t_hbm.at[pi_v.at[pl.ds(ch * C, C)]], prv, sem).start()

    def wait_gather(ch, crv, prv, sem):
        pltpu.make_async_copy(
            probs_t_hbm.at[ci_v.at[pl.ds(ch * C, C)]], crv, sem).wait()
        pltpu.make_async_copy(
            probs_t_hbm.at[pi_v.at[pl.ds(ch * C, C)]], prv, sem).wait()

    def chunk_compute(crv, prv, accs):
        zero = jnp.zeros((16,), jnp.float32)

        def edge_body(e, accs):
            accs = list(accs)
            for i in range(VPR):
                c = crv[e, pl.ds(i * 16, 16)]
                p = prv[e, pl.ds(i * 16, 16)]
                accs[i % 4] = accs[i % 4] + jnp.maximum(c - p, zero)
            return tuple(accs)

        return lax.fori_loop(0, C, edge_body, accs)

    start_gather(0, cr0_v, pr0_v, sem0)
    zero = jnp.zeros((16,), jnp.float32)

    def pair_body(j, accs):
        start_gather(2 * j + 1, cr1_v, pr1_v, sem1)
        wait_gather(2 * j, cr0_v, pr0_v, sem0)
        accs = chunk_compute(cr0_v, pr0_v, accs)
        start_gather(2 * j + 2, cr0_v, pr0_v, sem0)
        wait_gather(2 * j + 1, cr1_v, pr1_v, sem1)
        accs = chunk_compute(cr1_v, pr1_v, accs)
        return accs

    accs = lax.fori_loop(0, (N_CH - 1) // 2, pair_body,
                         (zero, zero, zero, zero))
    wait_gather(N_CH - 1, cr0_v, pr0_v, sem0)
    accs = chunk_compute(cr0_v, pr0_v, accs)

    out_v[...] = accs[0] + accs[1] + (accs[2] + accs[3])
    pltpu.sync_copy(out_v, out_hbm.at[wid])


@jax.jit
def _hierarchical_loss(probs, child, parent):
    probs_t = jnp.swapaxes(probs, 0, 1)
    pad = jnp.zeros((EP - E,), jnp.int32)
    child = jnp.concatenate([child, pad])
    parent = jnp.concatenate([parent, pad])
    mesh = plsc.VectorSubcoreMesh(core_axis_name="c", subcore_axis_name="s",
                                  num_cores=NC, num_subcores=NS)
    partials = pl.kernel(
        _sc_kernel,
        out_type=jax.ShapeDtypeStruct((NW, 16), jnp.float32),
        mesh=mesh,
        compiler_params=pltpu.CompilerParams(needs_layout_passes=False),
        scratch_types=[
            pltpu.VMEM((E_PER_W,), jnp.int32),
            pltpu.VMEM((E_PER_W,), jnp.int32),
            pltpu.VMEM((C, B), jnp.float32),
            pltpu.VMEM((C, B), jnp.float32),
            pltpu.VMEM((C, B), jnp.float32),
            pltpu.VMEM((C, B), jnp.float32),
            pltpu.VMEM((16,), jnp.float32),
            pltpu.SemaphoreType.DMA,
            pltpu.SemaphoreType.DMA,
        ],
    )(probs_t, child, parent)
    return jnp.sum(partials)


def kernel(probs, edge_index):
    child = edge_index[0].astype(jnp.int32)
    parent = edge_index[1].astype(jnp.int32)
    return _hierarchical_loss(probs, child, parent)


# speedup vs baseline: 1.7313x; 1.7313x over previous
import jax
import jax.numpy as jnp
from jax import lax
from jax.experimental import pallas as pl
from jax.experimental.pallas import tpu as pltpu
from jax.experimental.pallas import tpu_sc as plsc

B = 512
N = 45000
E = 100000
NC = 2
NS = 16
NW = NC * NS
PK = B // 2
PK_PER_W = PK // NW
N_PASS = PK_PER_W // 2
CHUNK = 4000
N_CHUNKS = E // CHUNK
VECS = CHUNK // 16


def _sc_kernel(packed_hbm, child_hbm, parent_hbm, out_hbm,
               pkA_v, pkB_v, ci0_v, pi0_v, ci1_v, pi1_v, out_v,
               sem0, sem1, semr):
    wid = lax.axis_index("s") * NC + lax.axis_index("c")
    pk_base = wid * PK_PER_W

    def start_idx(ci_v, pi_v, sem, ch):
        off = ch * CHUNK
        pltpu.make_async_copy(child_hbm.at[pl.ds(off, CHUNK)], ci_v, sem).start()
        pltpu.make_async_copy(parent_hbm.at[pl.ds(off, CHUNK)], pi_v, sem).start()

    def wait_idx(ci_v, pi_v, sem):
        pltpu.make_async_copy(child_hbm.at[pl.ds(0, CHUNK)], ci_v, sem).wait()
        pltpu.make_async_copy(parent_hbm.at[pl.ds(0, CHUNK)], pi_v, sem).wait()

    def chunk_compute(ci_v, pi_v, accs):
        def vec_body(i, accs):
            acc0, acc1 = accs
            ci = ci_v[pl.ds(i * 16, 16)]
            pi = pi_v[pl.ds(i * 16, 16)]
            cwA = plsc.load_gather(pkA_v, [ci])
            pwA = plsc.load_gather(pkA_v, [pi])
            cwB = plsc.load_gather(pkB_v, [ci])
            pwB = plsc.load_gather(pkB_v, [pi])
            zero = jnp.zeros((16,), jnp.float32)
            dA_hi = plsc.bitcast(cwA, jnp.float32) - plsc.bitcast(pwA, jnp.float32)
            dA_lo = plsc.bitcast(cwA << 16, jnp.float32) - plsc.bitcast(pwA << 16, jnp.float32)
            dB_hi = plsc.bitcast(cwB, jnp.float32) - plsc.bitcast(pwB, jnp.float32)
            dB_lo = plsc.bitcast(cwB << 16, jnp.float32) - plsc.bitcast(pwB << 16, jnp.float32)
            acc0 = acc0 + jnp.maximum(dA_hi, zero) + jnp.maximum(dB_hi, zero)
            acc1 = acc1 + jnp.maximum(dA_lo, zero) + jnp.maximum(dB_lo, zero)
            return acc0, acc1

        return lax.fori_loop(0, VECS, vec_body, accs)

    zero = jnp.zeros((16,), jnp.float32)
    accs = (zero, zero)
    for pp in range(N_PASS):
        r0 = pk_base + 2 * pp
        pltpu.make_async_copy(packed_hbm.at[r0], pkA_v, semr).start()
        pltpu.make_async_copy(packed_hbm.at[r0 + 1], pkB_v, semr).start()
        pltpu.make_async_copy(packed_hbm.at[0], pkA_v, semr).wait()
        pltpu.make_async_copy(packed_hbm.at[0], pkB_v, semr).wait()

        start_idx(ci0_v, pi0_v, sem0, 0)

        def pair_body(j, accs):
            start_idx(ci1_v, pi1_v, sem1, 2 * j + 1)
            wait_idx(ci0_v, pi0_v, sem0)
            accs = chunk_compute(ci0_v, pi0_v, accs)
            start_idx(ci0_v, pi0_v, sem0, 2 * j + 2)
            wait_idx(ci1_v, pi1_v, sem1)
            accs = chunk_compute(ci1_v, pi1_v, accs)
            return accs

        accs = lax.fori_loop(0, (N_CHUNKS - 1) // 2, pair_body, accs)
        wait_idx(ci0_v, pi0_v, sem0)
        accs = chunk_compute(ci0_v, pi0_v, accs)

    acc0, acc1 = accs
    out_v[...] = acc0 + acc1
    pltpu.sync_copy(out_v, out_hbm.at[wid])


PACK_BLK = 3000


def _pack_tc(pt_ref, out_ref):
    b = jax.lax.bitcast_convert_type(pt_ref[...], jnp.uint32)
    hi = b[:, PK:] & jnp.uint32(0xFFFF0000)
    lo = b[:, :PK] >> 16
    out_ref[...] = jax.lax.bitcast_convert_type(hi | lo, jnp.int32)


@jax.jit
def _hierarchical_loss(probs, child, parent):
    probs_t = jnp.swapaxes(probs, 0, 1)
    packed_nm = pl.pallas_call(
        _pack_tc,
        grid=(N // PACK_BLK,),
        in_specs=[pl.BlockSpec((PACK_BLK, B), lambda i: (i, 0))],
        out_specs=pl.BlockSpec((PACK_BLK, PK), lambda i: (i, 0)),
        out_shape=jax.ShapeDtypeStruct((N, PK), jnp.int32),
    )(probs_t)
    packed = jnp.swapaxes(packed_nm, 0, 1)

    mesh = plsc.VectorSubcoreMesh(core_axis_name="c", subcore_axis_name="s",
                                  num_cores=NC, num_subcores=NS)
    partials = pl.kernel(
        _sc_kernel,
        out_type=jax.ShapeDtypeStruct((NW, 16), jnp.float32),
        mesh=mesh,
        compiler_params=pltpu.CompilerParams(needs_layout_passes=False),
        scratch_types=[
            pltpu.VMEM((N,), jnp.int32),
            pltpu.VMEM((N,), jnp.int32),
            pltpu.VMEM((CHUNK,), jnp.int32),
            pltpu.VMEM((CHUNK,), jnp.int32),
            pltpu.VMEM((CHUNK,), jnp.int32),
            pltpu.VMEM((CHUNK,), jnp.int32),
            pltpu.VMEM((16,), jnp.float32),
            pltpu.SemaphoreType.DMA,
            pltpu.SemaphoreType.DMA,
            pltpu.SemaphoreType.DMA,
        ],
    )(packed, child, parent)
    return jnp.sum(partials)


def kernel(probs, edge_index):
    child = edge_index[0].astype(jnp.int32)
    parent = edge_index[1].astype(jnp.int32)
    return _hierarchical_loss(probs, child, parent)


# speedup vs baseline: 1.7581x; 1.0154x over previous
import jax
import jax.numpy as jnp
from jax import lax
from jax.experimental import pallas as pl
from jax.experimental.pallas import tpu as pltpu
from jax.experimental.pallas import tpu_sc as plsc

B = 512
N = 45000
E = 100000
NC = 2
NS = 16
NW = NC * NS
PK = B // 2
PK_PER_W = PK // NW
N_PASS = PK_PER_W // 2
CHUNK = 4000
N_CHUNKS = E // CHUNK
VECS = CHUNK // 16


def _sc_kernel(packed_hbm, child_hbm, parent_hbm, out_hbm,
               pkA_v, pkB_v, ci0_v, pi0_v, ci1_v, pi1_v, out_v,
               child_sp, parent_sp, sem0, sem1, semr):
    wid = lax.axis_index("s") * NC + lax.axis_index("c")
    pk_base = wid * PK_PER_W

    @pl.when(lax.axis_index("s") == 0)
    def _():
        pltpu.sync_copy(child_hbm, child_sp)
        pltpu.sync_copy(parent_hbm, parent_sp)
    plsc.subcore_barrier()

    def start_idx(ci_v, pi_v, sem, ch):
        off = ch * CHUNK
        pltpu.make_async_copy(child_sp.at[pl.ds(off, CHUNK)], ci_v, sem).start()
        pltpu.make_async_copy(parent_sp.at[pl.ds(off, CHUNK)], pi_v, sem).start()

    def wait_idx(ci_v, pi_v, sem):
        pltpu.make_async_copy(child_sp.at[pl.ds(0, CHUNK)], ci_v, sem).wait()
        pltpu.make_async_copy(parent_sp.at[pl.ds(0, CHUNK)], pi_v, sem).wait()

    def chunk_compute(ci_v, pi_v, accs):
        def vec_body(i, accs):
            acc0, acc1 = accs
            ci = ci_v[pl.ds(i * 16, 16)]
            pi = pi_v[pl.ds(i * 16, 16)]
            cwA = plsc.load_gather(pkA_v, [ci])
            pwA = plsc.load_gather(pkA_v, [pi])
            cwB = plsc.load_gather(pkB_v, [ci])
            pwB = plsc.load_gather(pkB_v, [pi])
            zero = jnp.zeros((16,), jnp.float32)
            dA_hi = plsc.bitcast(cwA, jnp.float32) - plsc.bitcast(pwA, jnp.float32)
            dA_lo = plsc.bitcast(cwA << 16, jnp.float32) - plsc.bitcast(pwA << 16, jnp.float32)
            dB_hi = plsc.bitcast(cwB, jnp.float32) - plsc.bitcast(pwB, jnp.float32)
            dB_lo = plsc.bitcast(cwB << 16, jnp.float32) - plsc.bitcast(pwB << 16, jnp.float32)
            acc0 = acc0 + jnp.maximum(dA_hi, zero) + jnp.maximum(dB_hi, zero)
            acc1 = acc1 + jnp.maximum(dA_lo, zero) + jnp.maximum(dB_lo, zero)
            return acc0, acc1

        return lax.fori_loop(0, VECS, vec_body, accs)

    zero = jnp.zeros((16,), jnp.float32)
    accs = (zero, zero)
    for pp in range(N_PASS):
        r0 = pk_base + 2 * pp
        pltpu.make_async_copy(packed_hbm.at[r0], pkA_v, semr).start()
        pltpu.make_async_copy(packed_hbm.at[r0 + 1], pkB_v, semr).start()
        pltpu.make_async_copy(packed_hbm.at[0], pkA_v, semr).wait()
        pltpu.make_async_copy(packed_hbm.at[0], pkB_v, semr).wait()

        start_idx(ci0_v, pi0_v, sem0, 0)

        def pair_body(j, accs):
            start_idx(ci1_v, pi1_v, sem1, 2 * j + 1)
            wait_idx(ci0_v, pi0_v, sem0)
            accs = chunk_compute(ci0_v, pi0_v, accs)
            start_idx(ci0_v, pi0_v, sem0, 2 * j + 2)
            wait_idx(ci1_v, pi1_v, sem1)
            accs = chunk_compute(ci1_v, pi1_v, accs)
            return accs

        accs = lax.fori_loop(0, (N_CHUNKS - 1) // 2, pair_body, accs)
        wait_idx(ci0_v, pi0_v, sem0)
        accs = chunk_compute(ci0_v, pi0_v, accs)

    acc0, acc1 = accs
    out_v[...] = acc0 + acc1
    pltpu.sync_copy(out_v, out_hbm.at[wid])


PACK_BLK = 3000


def _pack_tc(pt_ref, out_ref):
    b = jax.lax.bitcast_convert_type(pt_ref[...], jnp.uint32)
    hi = b[:, PK:] & jnp.uint32(0xFFFF0000)
    lo = b[:, :PK] >> 16
    out_ref[...] = jax.lax.bitcast_convert_type(hi | lo, jnp.int32)


@jax.jit
def _hierarchical_loss(probs, child, parent):
    probs_t = jnp.swapaxes(probs, 0, 1)
    packed_nm = pl.pallas_call(
        _pack_tc,
        grid=(N // PACK_BLK,),
        in_specs=[pl.BlockSpec((PACK_BLK, B), lambda i: (i, 0))],
        out_specs=pl.BlockSpec((PACK_BLK, PK), lambda i: (i, 0)),
        out_shape=jax.ShapeDtypeStruct((N, PK), jnp.int32),
    )(probs_t)
    packed = jnp.swapaxes(packed_nm, 0, 1)

    mesh = plsc.VectorSubcoreMesh(core_axis_name="c", subcore_axis_name="s",
                                  num_cores=NC, num_subcores=NS)
    partials = pl.kernel(
        _sc_kernel,
        out_type=jax.ShapeDtypeStruct((NW, 16), jnp.float32),
        mesh=mesh,
        compiler_params=pltpu.CompilerParams(needs_layout_passes=False),
        scratch_types=[
            pltpu.VMEM((N,), jnp.int32),
            pltpu.VMEM((N,), jnp.int32),
            pltpu.VMEM((CHUNK,), jnp.int32),
            pltpu.VMEM((CHUNK,), jnp.int32),
            pltpu.VMEM((CHUNK,), jnp.int32),
            pltpu.VMEM((CHUNK,), jnp.int32),
            pltpu.VMEM((16,), jnp.float32),
            pltpu.VMEM_SHARED((E,), jnp.int32),
            pltpu.VMEM_SHARED((E,), jnp.int32),
            pltpu.SemaphoreType.DMA,
            pltpu.SemaphoreType.DMA,
            pltpu.SemaphoreType.DMA,
        ],
    )(packed, child, parent)
    return jnp.sum(partials)


def kernel(probs, edge_index):
    child = edge_index[0].astype(jnp.int32)
    parent = edge_index[1].astype(jnp.int32)
    return _hierarchical_loss(probs, child, parent)
